# baseline (speedup 1.0000x reference)
import jax
import jax.numpy as jnp
from jax import lax
from jax.experimental import pallas as pl
from jax.experimental.pallas import tpu as pltpu
from jax.experimental.pallas import tpu_sc as plsc

S, P, O, V = 4, 8192, 2048, 2048
B = S * P
NC, NS = 2, 16
NW = NC * NS
BPW = B // NW
CG = 128
NCH = BPW // CG
NSLOT = 3
NWRD = 128
NBIT = O // NWRD
PB = 512
NBT = B // PB


def _sc_gather_body(words_hbm, seq_hbm, gout_hbm, idx_all, buf, gsem, osem):
    wid = lax.axis_index("s") * NC + lax.axis_index("c")
    base = wid * BPW
    pltpu.sync_copy(seq_hbm.at[pl.ds(base, BPW)], idx_all)

    def start_gather(k):
        b = lax.rem(k, NSLOT)
        pltpu.async_copy(words_hbm.at[idx_all.at[pl.ds(k * CG, CG)]],
                         buf.at[b], gsem.at[b])

    def wait_gather(b):
        pltpu.make_async_copy(words_hbm.at[idx_all.at[pl.ds(0, CG)]],
                              buf.at[b], gsem.at[b]).wait()

    def wait_scatter(b):
        pltpu.make_async_copy(buf.at[b], gout_hbm.at[pl.ds(0, CG), :],
                              osem.at[b]).wait()

    start_gather(0)
    start_gather(1)

    @pl.loop(0, NCH)
    def _chunk(k):
        b = lax.rem(k, NSLOT)

        @pl.when(k + 2 < NCH)
        def _():
            bn = lax.rem(k + 2, NSLOT)

            @pl.when(k >= 1)
            def _():
                wait_scatter(bn)
            start_gather(k + 2)

        wait_gather(b)
        pltpu.async_copy(buf.at[b], gout_hbm.at[pl.ds(base + k * CG, CG), :],
                         osem.at[b])

    for t in range(max(0, NCH - 3), NCH):
        wait_scatter(t % NSLOT)


def _tc_body(words_ref, logits_ref, out_ref):
    w = words_ref[...]
    rep = pltpu.repeat(w, NBIT, axis=1)
    k = lax.broadcasted_iota(jnp.int32, (PB, O), 1) // NWRD
    bit = (rep >> k) & 1
    out_ref[...] = jnp.where(bit != 0, -jnp.inf, logits_ref[...])


@jax.jit
def kernel(logits_SPT, seq_SP, valid_output_mask_TiTo):
    logits = logits_SPT.reshape(B, O).astype(jnp.float32)
    seq = seq_SP.reshape(B).astype(jnp.int32)

    b01 = jnp.isneginf(valid_output_mask_TiTo).astype(jnp.int32)
    weights = (jnp.int32(1) << jnp.arange(NBIT, dtype=jnp.int32))
    words = jnp.sum(b01.reshape(V, NBIT, NWRD) * weights[None, :, None],
                    axis=1)

    sc_gather = pl.kernel(
        _sc_gather_body,
        out_type=jax.ShapeDtypeStruct((B, NWRD), jnp.int32),
        mesh=plsc.VectorSubcoreMesh(
            core_axis_name="c", subcore_axis_name="s",
            num_cores=NC, num_subcores=NS),
        scratch_types=[
            pltpu.VMEM((BPW,), jnp.int32),
            pltpu.VMEM((NSLOT, CG, NWRD), jnp.int32),
            pltpu.SemaphoreType.DMA((NSLOT,)),
            pltpu.SemaphoreType.DMA((NSLOT,)),
        ],
    )
    gathered = sc_gather(words, seq)

    out = pl.pallas_call(
        _tc_body,
        grid=(NBT,),
        in_specs=[
            pl.BlockSpec((PB, NWRD), lambda i: (i, 0)),
            pl.BlockSpec((PB, O), lambda i: (i, 0)),
        ],
        out_specs=pl.BlockSpec((PB, O), lambda i: (i, 0)),
        out_shape=jax.ShapeDtypeStruct((B, O), jnp.float32),
    )(gathered, logits)
    return out.reshape(S, P, O)
